# Initial kernel scaffold; baseline (speedup 1.0000x reference)
import jax
import jax.numpy as jnp
from jax.experimental import pallas as pl


def kernel(features, edge_index, W1, Wh, W2, b2):
    raise NotImplementedError("write your pallas kernel here")


# speedup vs baseline: 4.2864x; 4.2864x over previous
import functools

import jax
import jax.numpy as jnp
from jax import lax
from jax.experimental import pallas as pl
from jax.experimental.pallas import tpu as pltpu
from jax.experimental.pallas import tpu_sc as plsc

_N = 10000
_NPAD = 10240
_E = 320000
_NC = 2
_NS = 16
_NW = _NC * _NS
_CH = 128
_K = 79
_EPAD = _NW * _K * _CH
_RPT = _NPAD // _NS
_D = 128
_DOUT = 64
_BLK = 512
_GRID = _NPAD // _BLK


def _mesh():
    return plsc.VectorSubcoreMesh(core_axis_name="c", subcore_axis_name="s",
                                  num_cores=_NC, num_subcores=_NS)


def _make_agg(d):
    out_type = [jax.ShapeDtypeStruct((_NC, _NPAD, d), jnp.float32)]
    scratch = [
        pltpu.VMEM((_K, _CH), jnp.int32),
        pltpu.VMEM((_K, _CH), jnp.int32),
        pltpu.VMEM((_CH, d), jnp.float32),
        pltpu.VMEM_SHARED((_NPAD, d), jnp.float32),
        pltpu.SemaphoreType.DMA,
    ]

    def body(h_hbm, src_hbm, dst_hbm, zacc_hbm, out_hbm,
             src_v, dst_v, rows_v, acc_s, sem):
        c = lax.axis_index("c")
        s = lax.axis_index("s")
        w = c * _NS + s
        r0 = s * _RPT
        pltpu.sync_copy(src_hbm.at[w], src_v)
        pltpu.sync_copy(dst_hbm.at[w], dst_v)
        pltpu.sync_copy(zacc_hbm, acc_s.at[pl.ds(r0, _RPT)])
        plsc.subcore_barrier()

        def step(j, carry):
            pltpu.async_copy(h_hbm.at[src_v.at[j]], rows_v, sem).wait()
            pltpu.sync_copy(rows_v, acc_s.at[dst_v.at[j]], add=True)
            return carry

        lax.fori_loop(0, _K, step, 0)
        plsc.subcore_barrier()
        pltpu.sync_copy(acc_s.at[pl.ds(r0, _RPT)],
                        out_hbm.at[c, pl.ds(r0, _RPT)])

    return pl.kernel(body, out_type=out_type, mesh=_mesh(),
                     scratch_types=scratch)


def _make_deg():
    out_type = [jax.ShapeDtypeStruct((_NC, _NPAD, _D), jnp.float32)]
    scratch = [
        pltpu.VMEM((_K, _CH), jnp.int32),
        pltpu.VMEM((_CH, _D), jnp.float32),
        pltpu.VMEM_SHARED((_NPAD, _D), jnp.float32),
    ]

    def body(idx_hbm, zacc_hbm, ones_hbm, deg_hbm, idx_v, ones_v, deg_s):
        c = lax.axis_index("c")
        s = lax.axis_index("s")
        w = c * _NS + s
        r0 = s * _RPT
        pltpu.sync_copy(idx_hbm.at[w], idx_v)
        pltpu.sync_copy(ones_hbm, ones_v)
        pltpu.sync_copy(zacc_hbm, deg_s.at[pl.ds(r0, _RPT)])
        plsc.subcore_barrier()

        def step(j, carry):
            pltpu.sync_copy(ones_v, deg_s.at[idx_v.at[j]], add=True)
            return carry

        lax.fori_loop(0, _K, step, 0)
        plsc.subcore_barrier()
        pltpu.sync_copy(deg_s.at[pl.ds(r0, _RPT)],
                        deg_hbm.at[c, pl.ds(r0, _RPT)])

    return pl.kernel(body, out_type=out_type, mesh=_mesh(),
                     scratch_types=scratch)


def _spec_rows(d):
    return pl.BlockSpec((_BLK, d), lambda i: (i, 0))


def _spec_part(p, d):
    return pl.BlockSpec((1, _BLK, d), lambda i, _p=p: (_p, i, 0))


def _spec_full(r, c):
    return pl.BlockSpec((r, c), lambda i: (0, 0))


def _rsq(a_ref, b_ref):
    deg = a_ref[0, :, :1] + b_ref[0, :, :1]
    return lax.rsqrt(jnp.maximum(deg, 1.0))


def _tc_mm(x, w):
    def body(x_ref, w_ref, o_ref):
        o_ref[...] = jnp.dot(x_ref[...], w_ref[...],
                             preferred_element_type=jnp.float32)
    return pl.pallas_call(
        body, grid=(_GRID,),
        in_specs=[_spec_rows(_D), _spec_full(_D, _D)],
        out_specs=_spec_rows(_D),
        out_shape=jax.ShapeDtypeStruct((_NPAD, _D), jnp.float32),
    )(x, w)


def _tc_l2(p1, dego, wh):
    def body(pa, pb, da, db, w_ref, o_ref):
        x2 = jnp.maximum(pa[0] + pb[0], 0.0) * _rsq(da, db)
        o_ref[...] = jnp.dot(x2, w_ref[...],
                             preferred_element_type=jnp.float32)
    return pl.pallas_call(
        body, grid=(_GRID,),
        in_specs=[_spec_part(0, _D), _spec_part(1, _D),
                  _spec_part(0, _D), _spec_part(1, _D),
                  _spec_full(_D, _D)],
        out_specs=_spec_rows(_D),
        out_shape=jax.ShapeDtypeStruct((_NPAD, _D), jnp.float32),
    )(p1, p1, dego, dego, wh)


def _tc_l3(p2, dego, degi, w2):
    def body(pa, pb, doa, dob, dia, dib, w_ref, o_ref):
        x3 = jnp.maximum((pa[0] + pb[0]) * _rsq(dia, dib), 0.0)
        o_ref[...] = jnp.dot(x3 * _rsq(doa, dob), w_ref[...],
                             preferred_element_type=jnp.float32)
    return pl.pallas_call(
        body, grid=(_GRID,),
        in_specs=[_spec_part(0, _D), _spec_part(1, _D),
                  _spec_part(0, _D), _spec_part(1, _D),
                  _spec_part(0, _D), _spec_part(1, _D),
                  _spec_full(_D, _D)],
        out_specs=_spec_rows(_D),
        out_shape=jax.ShapeDtypeStruct((_NPAD, _D), jnp.float32),
    )(p2, p2, dego, dego, degi, degi, w2)


def _tc_l4(p3, degi, b2):
    def body(pa, pb, dia, dib, b_ref, o_ref):
        y = (pa[0, :, :_DOUT] + pb[0, :, :_DOUT]) * _rsq(dia, dib) + b_ref[...]
        m = jnp.max(y, axis=1, keepdims=True)
        z = y - m
        o_ref[...] = z - jnp.log(jnp.sum(jnp.exp(z), axis=1, keepdims=True))
    return pl.pallas_call(
        body, grid=(_GRID,),
        in_specs=[_spec_part(0, _D), _spec_part(1, _D),
                  _spec_part(0, _D), _spec_part(1, _D),
                  _spec_full(1, _DOUT)],
        out_specs=_spec_rows(_DOUT),
        out_shape=jax.ShapeDtypeStruct((_NPAD, _DOUT), jnp.float32),
    )(p3, p3, degi, degi, b2)


_deg = _make_deg()
_agg128 = _make_agg(_D)


def kernel(features, edge_index, W1, Wh, W2, b2):
    f32 = jnp.float32
    x = jnp.zeros((_NPAD, _D), f32).at[:_N].set(features)
    pad = jnp.full((2, _EPAD - _E), _N, jnp.int32)
    ei = jnp.concatenate([edge_index.astype(jnp.int32), pad], axis=1)
    src = ei[0].reshape(_NW, _K, _CH)
    dst = ei[1].reshape(_NW, _K, _CH)
    zacc = jnp.zeros((_RPT, _D), f32)
    ones = jnp.ones((_CH, _D), f32)
    w2p = jnp.zeros((_D, _D), f32).at[:, :_DOUT].set(W2)

    h1 = _tc_mm(x, W1)
    (dego,) = _deg(src, zacc, ones)
    (degi,) = _deg(dst, zacc, ones)
    (p1,) = _agg128(h1, src, dst, zacc)
    h2 = _tc_l2(p1, dego, Wh)
    (p2,) = _agg128(h2, src, dst, zacc)
    h3 = _tc_l3(p2, dego, degi, w2p)
    (p3,) = _agg128(h3, src, dst, zacc)
    y = _tc_l4(p3, degi, b2.reshape(1, _DOUT))
    return y[:_N]
